# baseline (device time: 147901 ns/iter reference)
import jax
import jax.numpy as jnp
from jax import lax
from jax.experimental import pallas as pl
from jax.experimental.pallas import tpu as pltpu

N_DEV = 32
N_TOK = 512
D_IN = 256
D_OUT = 512
N_EXP = 64
E_PER = 2
CH = N_TOK // N_DEV
N_HOPS = 2 * (N_DEV - 1)


def kernel(x, router_W, route_idx, expert_W, shared_W):
    def body(x_ref, rw_ref, idx_ref, ew_ref, sw_ref, out_ref,
             acc_ref, comm_ref, send_sems, recv_sems):
        my = lax.axis_index("i")
        left = lax.rem(my + N_DEV - 1, N_DEV)
        right = lax.rem(my + 1, N_DEV)

        barrier_sem = pltpu.get_barrier_semaphore()
        pl.semaphore_signal(barrier_sem, inc=1, device_id=(left,),
                            device_id_type=pl.DeviceIdType.MESH)
        pl.semaphore_signal(barrier_sem, inc=1, device_id=(right,),
                            device_id_type=pl.DeviceIdType.MESH)
        pl.semaphore_wait(barrier_sem, 2)

        x = x_ref[:, :]
        scores = jnp.dot(x, rw_ref[:, :], preferred_element_type=jnp.float32)
        m = jnp.max(scores, axis=-1, keepdims=True)
        e = jnp.exp(scores - m)
        probs = e / jnp.sum(e, axis=-1, keepdims=True)
        idx = idx_ref[:, :]
        col = lax.broadcasted_iota(jnp.int32, (N_TOK, N_EXP), 1)
        acc = None
        for k in range(E_PER):
            gl = my * E_PER + k
            p_gl = jnp.sum(jnp.where(col == gl, probs, 0.0), axis=-1,
                           keepdims=True)
            w = jnp.where(idx == gl, p_gl, 0.0)
            y = jnp.dot(x, ew_ref[k, :, :], preferred_element_type=jnp.float32)
            term = w * y
            acc = term if acc is None else acc + term
        acc_ref[:, :] = acc

        for s in range(N_DEV - 1):
            c_send = lax.rem(my - s + 2 * N_DEV, N_DEV)
            rdma = pltpu.make_async_remote_copy(
                src_ref=acc_ref.at[pl.ds(c_send * CH, CH)],
                dst_ref=comm_ref.at[s],
                send_sem=send_sems.at[s],
                recv_sem=recv_sems.at[s],
                device_id=(right,),
                device_id_type=pl.DeviceIdType.MESH,
            )
            rdma.start()
            rdma.wait()
            c_recv = lax.rem(my - s - 1 + 2 * N_DEV, N_DEV)
            r0 = c_recv * CH
            acc_ref[pl.ds(r0, CH), :] = acc_ref[pl.ds(r0, CH), :] + comm_ref[s, :, :]

        for s in range(N_DEV - 1):
            hop = N_DEV - 1 + s
            c_send = lax.rem(my + 1 - s + 2 * N_DEV, N_DEV)
            rdma = pltpu.make_async_remote_copy(
                src_ref=acc_ref.at[pl.ds(c_send * CH, CH)],
                dst_ref=comm_ref.at[hop],
                send_sem=send_sems.at[hop],
                recv_sem=recv_sems.at[hop],
                device_id=(right,),
                device_id_type=pl.DeviceIdType.MESH,
            )
            rdma.start()
            rdma.wait()
            c_recv = lax.rem(my - s + 2 * N_DEV, N_DEV)
            acc_ref[pl.ds(c_recv * CH, CH), :] = comm_ref[hop, :, :]

        out_ref[:, :] = acc_ref[:, :] + jnp.dot(
            x, sw_ref[:, :], preferred_element_type=jnp.float32)

    return pl.pallas_call(
        body,
        out_shape=jax.ShapeDtypeStruct((N_TOK, D_OUT), jnp.float32),
        in_specs=[pl.BlockSpec(memory_space=pltpu.VMEM)] * 5,
        out_specs=pl.BlockSpec(memory_space=pltpu.VMEM),
        scratch_shapes=[
            pltpu.VMEM((N_TOK, D_OUT), jnp.float32),
            pltpu.VMEM((N_HOPS, CH, D_OUT), jnp.float32),
            pltpu.SemaphoreType.DMA((N_HOPS,)),
            pltpu.SemaphoreType.DMA((N_HOPS,)),
        ],
        compiler_params=pltpu.CompilerParams(collective_id=0),
    )(x, router_W, route_idx, expert_W, shared_W)


# device time: 46790 ns/iter; 3.1610x vs baseline; 3.1610x over previous
import jax
import jax.numpy as jnp
from jax import lax
from jax.experimental import pallas as pl
from jax.experimental.pallas import tpu as pltpu

N_DEV = 32
N_TOK = 512
D_IN = 256
D_OUT = 512
N_EXP = 64
E_PER = 2
CH = N_TOK // N_DEV


def kernel(x, router_W, route_idx, expert_W, shared_W):
    def body(x_ref, rw_ref, idx_ref, ew_ref, sw_ref, out_ref,
             acc_ref, rs_ref, ssA, rsA, ssB, rsB):
        my = lax.axis_index("i")

        x = x_ref[:, :]
        scores = jnp.dot(x, rw_ref[:, :], preferred_element_type=jnp.float32)
        m = jnp.max(scores, axis=-1, keepdims=True)
        e = jnp.exp(scores - m)
        probs = e / jnp.sum(e, axis=-1, keepdims=True)
        idx = idx_ref[:, :]
        col = lax.broadcasted_iota(jnp.int32, (N_TOK, N_EXP), 1)
        acc = None
        for k in range(E_PER):
            gl = my * E_PER + k
            p_gl = jnp.sum(jnp.where(col == gl, probs, 0.0), axis=-1,
                           keepdims=True)
            w = jnp.where(idx == gl, p_gl, 0.0)
            y = jnp.dot(x, ew_ref[k, :, :], preferred_element_type=jnp.float32)
            term = w * y
            acc = term if acc is None else acc + term
        acc_ref[:, :] = acc

        sendsA = []
        for k in range(1, N_DEV):
            tgt = lax.rem(my + k, N_DEV)
            rdma = pltpu.make_async_remote_copy(
                src_ref=acc_ref.at[pl.ds(tgt * CH, CH)],
                dst_ref=rs_ref.at[my],
                send_sem=ssA.at[k - 1],
                recv_sem=rsA.at[my],
                device_id=(tgt,),
                device_id_type=pl.DeviceIdType.MESH,
            )
            rdma.start()
            sendsA.append(rdma)

        rs_ref[my, :, :] = acc_ref[pl.ds(my * CH, CH), :]
        shared_y = jnp.dot(x, sw_ref[:, :], preferred_element_type=jnp.float32)

        for k in range(1, N_DEV):
            src = lax.rem(my + k, N_DEV)
            recv = pltpu.make_async_remote_copy(
                src_ref=rs_ref.at[src],
                dst_ref=rs_ref.at[src],
                send_sem=ssA.at[0],
                recv_sem=rsA.at[src],
                device_id=(src,),
                device_id_type=pl.DeviceIdType.MESH,
            )
            recv.wait_recv()

        reduced = jnp.sum(rs_ref[:, :, :], axis=0)
        out_ref[pl.ds(my * CH, CH), :] = reduced

        sendsB = []
        for k in range(1, N_DEV):
            tgt = lax.rem(my + k, N_DEV)
            rdma = pltpu.make_async_remote_copy(
                src_ref=out_ref.at[pl.ds(my * CH, CH)],
                dst_ref=out_ref.at[pl.ds(my * CH, CH)],
                send_sem=ssB.at[k - 1],
                recv_sem=rsB.at[my],
                device_id=(tgt,),
                device_id_type=pl.DeviceIdType.MESH,
            )
            rdma.start()
            sendsB.append(rdma)

        for rdma in sendsA:
            rdma.wait_send()

        for k in range(1, N_DEV):
            src = lax.rem(my + k, N_DEV)
            recv = pltpu.make_async_remote_copy(
                src_ref=out_ref.at[pl.ds(src * CH, CH)],
                dst_ref=out_ref.at[pl.ds(src * CH, CH)],
                send_sem=ssB.at[0],
                recv_sem=rsB.at[src],
                device_id=(src,),
                device_id_type=pl.DeviceIdType.MESH,
            )
            recv.wait_recv()
        for rdma in sendsB:
            rdma.wait_send()

        out_ref[:, :] = out_ref[:, :] + shared_y

    return pl.pallas_call(
        body,
        out_shape=jax.ShapeDtypeStruct((N_TOK, D_OUT), jnp.float32),
        in_specs=[pl.BlockSpec(memory_space=pltpu.VMEM)] * 5,
        out_specs=pl.BlockSpec(memory_space=pltpu.VMEM),
        scratch_shapes=[
            pltpu.VMEM((N_TOK, D_OUT), jnp.float32),
            pltpu.VMEM((N_DEV, CH, D_OUT), jnp.float32),
            pltpu.SemaphoreType.DMA((N_DEV,)),
            pltpu.SemaphoreType.DMA((N_DEV,)),
            pltpu.SemaphoreType.DMA((N_DEV,)),
            pltpu.SemaphoreType.DMA((N_DEV,)),
        ],
    )(x, router_W, route_idx, expert_W, shared_W)


# device time: 42151 ns/iter; 3.5088x vs baseline; 1.1101x over previous
import jax
import jax.numpy as jnp
from jax import lax
from jax.experimental import pallas as pl
from jax.experimental.pallas import tpu as pltpu

N_DEV = 32
N_TOK = 512
D_IN = 256
D_OUT = 512
N_EXP = 64
E_PER = 2
CH = N_TOK // N_DEV


def kernel(x, router_W, route_idx, expert_W, shared_W):
    def body(x_ref, rw_ref, idx_ref, ew_ref, sw_ref, out_ref,
             acc_ref, rs_ref, ssA, rsA, ssB, rsB):
        my = lax.axis_index("i")

        barrier_sem = pltpu.get_barrier_semaphore()
        for k in range(1, N_DEV):
            pl.semaphore_signal(barrier_sem, inc=1,
                                device_id=(lax.rem(my + k, N_DEV),),
                                device_id_type=pl.DeviceIdType.MESH)
        pl.semaphore_wait(barrier_sem, N_DEV - 1)

        x = x_ref[:, :]
        scores = jnp.dot(x, rw_ref[:, :], preferred_element_type=jnp.float32)
        m = jnp.max(scores, axis=-1, keepdims=True)
        e = jnp.exp(scores - m)
        probs = e / jnp.sum(e, axis=-1, keepdims=True)
        idx = idx_ref[:, :]
        col = lax.broadcasted_iota(jnp.int32, (N_TOK, N_EXP), 1)
        acc = None
        for k in range(E_PER):
            gl = my * E_PER + k
            p_gl = jnp.sum(jnp.where(col == gl, probs, 0.0), axis=-1,
                           keepdims=True)
            w = jnp.where(idx == gl, p_gl, 0.0)
            y = jnp.dot(x, ew_ref[k, :, :], preferred_element_type=jnp.float32)
            term = w * y
            acc = term if acc is None else acc + term
        acc_ref[:, :] = acc

        sendsA = []
        for k in range(1, N_DEV):
            tgt = lax.rem(my + k, N_DEV)
            rdma = pltpu.make_async_remote_copy(
                src_ref=acc_ref.at[pl.ds(tgt * CH, CH)],
                dst_ref=rs_ref.at[my],
                send_sem=ssA.at[k - 1],
                recv_sem=rsA.at[my],
                device_id=(tgt,),
                device_id_type=pl.DeviceIdType.MESH,
            )
            rdma.start()
            sendsA.append(rdma)

        rs_ref[my, :, :] = acc_ref[pl.ds(my * CH, CH), :]
        sh_mine = jnp.dot(x_ref[pl.ds(my * CH, CH), :], sw_ref[:, :],
                          preferred_element_type=jnp.float32)

        for k in range(1, N_DEV):
            src = lax.rem(my + k, N_DEV)
            recv = pltpu.make_async_remote_copy(
                src_ref=rs_ref.at[src],
                dst_ref=rs_ref.at[src],
                send_sem=ssA.at[0],
                recv_sem=rsA.at[src],
                device_id=(src,),
                device_id_type=pl.DeviceIdType.MESH,
            )
            recv.wait_recv()

        reduced = jnp.sum(rs_ref[:, :, :], axis=0)
        out_ref[pl.ds(my * CH, CH), :] = reduced + sh_mine

        sendsB = []
        for k in range(1, N_DEV):
            tgt = lax.rem(my + k, N_DEV)
            rdma = pltpu.make_async_remote_copy(
                src_ref=out_ref.at[pl.ds(my * CH, CH)],
                dst_ref=out_ref.at[pl.ds(my * CH, CH)],
                send_sem=ssB.at[k - 1],
                recv_sem=rsB.at[my],
                device_id=(tgt,),
                device_id_type=pl.DeviceIdType.MESH,
            )
            rdma.start()
            sendsB.append(rdma)

        for rdma in sendsA:
            rdma.wait_send()

        for k in range(1, N_DEV):
            src = lax.rem(my + k, N_DEV)
            recv = pltpu.make_async_remote_copy(
                src_ref=out_ref.at[pl.ds(src * CH, CH)],
                dst_ref=out_ref.at[pl.ds(src * CH, CH)],
                send_sem=ssB.at[0],
                recv_sem=rsB.at[src],
                device_id=(src,),
                device_id_type=pl.DeviceIdType.MESH,
            )
            recv.wait_recv()
        for rdma in sendsB:
            rdma.wait_send()

    return pl.pallas_call(
        body,
        out_shape=jax.ShapeDtypeStruct((N_TOK, D_OUT), jnp.float32),
        in_specs=[pl.BlockSpec(memory_space=pltpu.VMEM)] * 5,
        out_specs=pl.BlockSpec(memory_space=pltpu.VMEM),
        scratch_shapes=[
            pltpu.VMEM((N_TOK, D_OUT), jnp.float32),
            pltpu.VMEM((N_DEV, CH, D_OUT), jnp.float32),
            pltpu.SemaphoreType.DMA((N_DEV,)),
            pltpu.SemaphoreType.DMA((N_DEV,)),
            pltpu.SemaphoreType.DMA((N_DEV,)),
            pltpu.SemaphoreType.DMA((N_DEV,)),
        ],
        compiler_params=pltpu.CompilerParams(collective_id=0),
    )(x, router_W, route_idx, expert_W, shared_W)


# device time: 42097 ns/iter; 3.5133x vs baseline; 1.0013x over previous
import os

import jax
import jax.numpy as jnp
from jax import lax
from jax.experimental import pallas as pl
from jax.experimental.pallas import tpu as pltpu

_VARIANT = os.environ.get("KERNEL_VARIANT", "full")
_HAS_BARRIER = _VARIANT in ("barrier", "phaseA", "full")
_HAS_A = _VARIANT in ("phaseA", "full")
_HAS_B = _VARIANT == "full"

N_DEV = 32
N_TOK = 512
D_IN = 256
D_OUT = 512
N_EXP = 64
E_PER = 2
CH = N_TOK // N_DEV


def kernel(x, router_W, route_idx, expert_W, shared_W):
    def body(x_ref, rw_ref, idx_ref, ew_ref, sw_ref, out_ref,
             acc_ref, rs_ref, ssA, rsA, ssB, rsB):
        my = lax.axis_index("i")

        if _HAS_BARRIER:
            barrier_sem = pltpu.get_barrier_semaphore()
            for k in range(1, N_DEV):
                pl.semaphore_signal(barrier_sem, inc=1,
                                    device_id=(lax.rem(my + k, N_DEV),),
                                    device_id_type=pl.DeviceIdType.MESH)
            pl.semaphore_wait(barrier_sem, N_DEV - 1)

        x = x_ref[:, :]
        scores = jnp.dot(x, rw_ref[:, :], preferred_element_type=jnp.float32)
        m = jnp.max(scores, axis=-1, keepdims=True)
        e = jnp.exp(scores - m)
        probs = e / jnp.sum(e, axis=-1, keepdims=True)
        idx = idx_ref[:, :]
        col = lax.broadcasted_iota(jnp.int32, (N_TOK, N_EXP), 1)
        acc = None
        for k in range(E_PER):
            gl = my * E_PER + k
            p_gl = jnp.sum(jnp.where(col == gl, probs, 0.0), axis=-1,
                           keepdims=True)
            w = jnp.where(idx == gl, p_gl, 0.0)
            y = jnp.dot(x, ew_ref[k, :, :], preferred_element_type=jnp.float32)
            term = w * y
            acc = term if acc is None else acc + term
        acc_ref[:, :] = acc

        sendsA = []
        for k in range(1, N_DEV) if _HAS_A else ():
            tgt = lax.rem(my + k, N_DEV)
            rdma = pltpu.make_async_remote_copy(
                src_ref=acc_ref.at[pl.ds(tgt * CH, CH)],
                dst_ref=rs_ref.at[my],
                send_sem=ssA.at[k - 1],
                recv_sem=rsA.at[my],
                device_id=(tgt,),
                device_id_type=pl.DeviceIdType.MESH,
            )
            rdma.start()
            sendsA.append(rdma)

        rs_ref[my, :, :] = acc_ref[pl.ds(my * CH, CH), :]
        sh_mine = jnp.dot(x_ref[pl.ds(my * CH, CH), :], sw_ref[:, :],
                          preferred_element_type=jnp.float32)

        for k in range(1, N_DEV) if _HAS_A else ():
            src = lax.rem(my + k, N_DEV)
            recv = pltpu.make_async_remote_copy(
                src_ref=rs_ref.at[src],
                dst_ref=rs_ref.at[src],
                send_sem=ssA.at[0],
                recv_sem=rsA.at[src],
                device_id=(src,),
                device_id_type=pl.DeviceIdType.MESH,
            )
            recv.wait_recv()

        reduced = jnp.sum(rs_ref[:, :, :], axis=0)
        out_ref[pl.ds(my * CH, CH), :] = reduced + sh_mine

        sendsB = []
        for k in range(1, N_DEV) if _HAS_B else ():
            tgt = lax.rem(my + k, N_DEV)
            rdma = pltpu.make_async_remote_copy(
                src_ref=out_ref.at[pl.ds(my * CH, CH)],
                dst_ref=out_ref.at[pl.ds(my * CH, CH)],
                send_sem=ssB.at[k - 1],
                recv_sem=rsB.at[my],
                device_id=(tgt,),
                device_id_type=pl.DeviceIdType.MESH,
            )
            rdma.start()
            sendsB.append(rdma)

        for rdma in sendsA:
            rdma.wait_send()

        for k in range(1, N_DEV) if _HAS_B else ():
            src = lax.rem(my + k, N_DEV)
            recv = pltpu.make_async_remote_copy(
                src_ref=out_ref.at[pl.ds(src * CH, CH)],
                dst_ref=out_ref.at[pl.ds(src * CH, CH)],
                send_sem=ssB.at[0],
                recv_sem=rsB.at[src],
                device_id=(src,),
                device_id_type=pl.DeviceIdType.MESH,
            )
            recv.wait_recv()
        for rdma in sendsB:
            rdma.wait_send()

    return pl.pallas_call(
        body,
        out_shape=jax.ShapeDtypeStruct((N_TOK, D_OUT), jnp.float32),
        in_specs=[pl.BlockSpec(memory_space=pltpu.VMEM)] * 5,
        out_specs=pl.BlockSpec(memory_space=pltpu.VMEM),
        scratch_shapes=[
            pltpu.VMEM((N_TOK, D_OUT), jnp.float32),
            pltpu.VMEM((N_DEV, CH, D_OUT), jnp.float32),
            pltpu.SemaphoreType.DMA((N_DEV,)),
            pltpu.SemaphoreType.DMA((N_DEV,)),
            pltpu.SemaphoreType.DMA((N_DEV,)),
            pltpu.SemaphoreType.DMA((N_DEV,)),
        ],
        compiler_params=(pltpu.CompilerParams(collective_id=0)
                         if _HAS_BARRIER else pltpu.CompilerParams()),
    )(x, router_W, route_idx, expert_W, shared_W)
